# Initial kernel scaffold; baseline (speedup 1.0000x reference)
import jax
import jax.numpy as jnp
from jax.experimental import pallas as pl


def kernel(z_e_x, W):
    raise NotImplementedError("write your pallas kernel here")


# speedup vs baseline: 1.1779x; 1.1779x over previous
import jax
import jax.numpy as jnp
from jax.experimental import pallas as pl

_BN = 256


def _vq_argmin_kernel(z_ref, w_ref, zsq_ref, wsq_ref, out_ref):
    z = z_ref[...]
    w = w_ref[...]
    lhs = (2.0 * z).astype(jnp.bfloat16)
    dot = jax.lax.dot_general(
        lhs, w, (((1,), (1,)), ((), ())),
        preferred_element_type=jnp.float32)
    dists = (zsq_ref[...] - dot) + wsq_ref[...]
    k = dists.shape[1]
    m = jnp.min(dists, axis=1, keepdims=True)
    iota = jax.lax.broadcasted_iota(jnp.int32, dists.shape, 1)
    idx = jnp.min(jnp.where(dists == m, iota, k), axis=1)
    out_ref[...] = idx.reshape(1, 1, -1)


def kernel(z_e_x, W):
    input_shape = z_e_x.shape
    d = input_shape[-1]
    flat = z_e_x.reshape(-1, d)
    n, k = flat.shape[0], W.shape[0]
    zsq = jnp.sum(flat ** 2, axis=1, keepdims=True)
    wsq = jnp.sum(W ** 2, axis=1)[None, :]

    grid = n // _BN
    out = pl.pallas_call(
        _vq_argmin_kernel,
        grid=(grid,),
        in_specs=[
            pl.BlockSpec((_BN, d), lambda i: (i, 0)),
            pl.BlockSpec((k, d), lambda i: (0, 0)),
            pl.BlockSpec((_BN, 1), lambda i: (i, 0)),
            pl.BlockSpec((1, k), lambda i: (0, 0)),
        ],
        out_specs=pl.BlockSpec((1, 1, _BN), lambda i: (i, 0, 0)),
        out_shape=jax.ShapeDtypeStruct((grid, 1, _BN), jnp.int32),
    )(flat, W, zsq, wsq)
    return out.reshape(input_shape[:-1])
